# Initial kernel scaffold; baseline (speedup 1.0000x reference)
import jax
import jax.numpy as jnp
from jax.experimental import pallas as pl


def kernel(x):
    raise NotImplementedError("write your pallas kernel here")


# speedup vs baseline: 5.6442x; 5.6442x over previous
import jax
import jax.numpy as jnp
from jax.experimental import pallas as pl
from jax.experimental.pallas import tpu as pltpu

BS = 128
NP = 1024
NV = 16
PL = 12
C = NV * PL
K_TIME = 512
K_FREQ = 409


def _body(x_ref, xt_ref, tm_ref, xf_ref, fm_ref, ut_ref):
    b = pl.program_id(0)

    @pl.when(b == 0)
    def _init_ut():
        r = jax.lax.broadcasted_iota(jnp.int32, (NP, NP), 0)
        c = jax.lax.broadcasted_iota(jnp.int32, (NP, NP), 1)
        ut_ref[...] = (r < c).astype(jnp.bfloat16)

    xb = x_ref[...]

    ci = jax.lax.broadcasted_iota(jnp.int32, (C, NV), 0)
    vi = jax.lax.broadcasted_iota(jnp.int32, (C, NV), 1)
    S = jnp.where(ci // PL == vi, 1.0, 0.0).astype(jnp.float32)

    sums = jnp.dot(xb, S, preferred_element_type=jnp.float32,
                   precision=jax.lax.Precision.HIGHEST)
    sumsq = jnp.dot(xb * xb, S, preferred_element_type=jnp.float32,
                    precision=jax.lax.Precision.HIGHEST)
    mean = sums * (1.0 / PL)
    var = (sumsq - sums * mean) * (1.0 / (PL - 1))
    var = jnp.maximum(var, 0.0)
    cv = jnp.sqrt(var) / (mean + 1e-6)

    cvt = cv.T
    bits1 = jax.lax.bitcast_convert_type(cvt, jnp.int32)
    bits = jnp.concatenate([bits1, bits1], axis=0)
    rowi = jax.lax.broadcasted_iota(jnp.int32, (2 * NV, 1), 0)
    kvec = jnp.where(rowi < NV, K_TIME, K_FREQ)

    t = jnp.zeros((2 * NV, 1), jnp.int32)
    for shift in range(30, -1, -1):
        cand = t | (1 << shift)
        cnt = jnp.sum((bits >= cand).astype(jnp.int32), axis=1, keepdims=True)
        t = jnp.where(cnt >= kvec, cand, t)

    gt = bits > t
    eq = bits == t
    g = jnp.sum(gt.astype(jnp.int32), axis=1, keepdims=True)
    need = (kvec - g).astype(jnp.float32)
    prefix = jnp.dot(eq.astype(jnp.bfloat16), ut_ref[...],
                     preferred_element_type=jnp.float32)
    tie = eq & (prefix < need)
    keep = 1.0 - (gt | tie).astype(jnp.float32)

    tmask = keep[:NV].T
    fmask = keep[NV:].T
    tm_ref[...] = tmask
    fm_ref[...] = fmask

    St = jnp.where(vi.T == ci.T // PL, 1.0, 0.0).astype(jnp.float32)
    xt_ref[...] = xb * jnp.dot(tmask, St, preferred_element_type=jnp.float32)
    xf_ref[...] = xb * jnp.dot(fmask, St, preferred_element_type=jnp.float32)


def _run(x2, interpret=False):
    return pl.pallas_call(
        _body,
        grid=(BS,),
        in_specs=[pl.BlockSpec((None, NP, C), lambda b: (b, 0, 0))],
        out_specs=(
            pl.BlockSpec((None, NP, C), lambda b: (b, 0, 0)),
            pl.BlockSpec((None, NP, NV), lambda b: (b, 0, 0)),
            pl.BlockSpec((None, NP, C), lambda b: (b, 0, 0)),
            pl.BlockSpec((None, NP, NV), lambda b: (b, 0, 0)),
        ),
        out_shape=(
            jax.ShapeDtypeStruct((BS, NP, C), jnp.float32),
            jax.ShapeDtypeStruct((BS, NP, NV), jnp.float32),
            jax.ShapeDtypeStruct((BS, NP, C), jnp.float32),
            jax.ShapeDtypeStruct((BS, NP, NV), jnp.float32),
        ),
        scratch_shapes=[pltpu.VMEM((NP, NP), jnp.bfloat16)],
        interpret=interpret,
    )(x2)


def kernel(x):
    bs, np_, nv, plen = x.shape
    x2 = x.reshape(bs, np_, nv * plen)
    xt, tm, xf, fm = _run(x2)
    return (xt.reshape(x.shape), tm, xf.reshape(x.shape), fm)


# speedup vs baseline: 16.6588x; 2.9515x over previous
import jax
import jax.numpy as jnp
from jax.experimental import pallas as pl
from jax.experimental.pallas import tpu as pltpu

BS = 128
NP = 1024
NV = 16
PLEN = 12
K_TIME = 512
K_FREQ = 409


def _body(x_ref, xt_ref, tm_ref, xf_ref, fm_ref, ut_ref):
    b = pl.program_id(0)

    @pl.when(b == 0)
    def _init_ut():
        r = jax.lax.broadcasted_iota(jnp.int32, (NP, NP), 0)
        c = jax.lax.broadcasted_iota(jnp.int32, (NP, NP), 1)
        ut_ref[...] = (r < c).astype(jnp.bfloat16)

    xb = x_ref[...]
    s = jnp.sum(xb, axis=0)
    sq = jnp.sum(xb * xb, axis=0)
    mean = s * (1.0 / PLEN)
    var = (sq - s * mean) * (1.0 / (PLEN - 1))
    var = jnp.maximum(var, 0.0)
    cv = jnp.sqrt(var) / (mean + 1e-6)

    bits1 = jax.lax.bitcast_convert_type(cv, jnp.int32)
    bits = jnp.concatenate([bits1, bits1], axis=0)
    rowi = jax.lax.broadcasted_iota(jnp.int32, (2 * NV, 1), 0)
    kvec = jnp.where(rowi < NV, K_TIME, K_FREQ)

    t = jnp.zeros((2 * NV, 1), jnp.int32)
    for shift in range(30, -1, -1):
        cand = t | (1 << shift)
        cnt = jnp.sum((bits >= cand).astype(jnp.int32), axis=1, keepdims=True)
        t = jnp.where(cnt >= kvec, cand, t)

    gt = bits > t
    eq = bits == t
    g = jnp.sum(gt.astype(jnp.int32), axis=1, keepdims=True)
    need = (kvec - g).astype(jnp.float32)
    prefix = jnp.dot(eq.astype(jnp.bfloat16), ut_ref[...],
                     preferred_element_type=jnp.float32)
    tie = eq & (prefix < need)
    keep = 1.0 - (gt | tie).astype(jnp.float32)

    tmask = keep[:NV]
    fmask = keep[NV:]
    tm_ref[...] = tmask
    fm_ref[...] = fmask
    xt_ref[...] = xb * tmask[None]
    xf_ref[...] = xb * fmask[None]


def _run(xt, interpret=False):
    bs = xt.shape[0]
    return pl.pallas_call(
        _body,
        grid=(bs,),
        in_specs=[pl.BlockSpec((None, PLEN, NV, NP), lambda b: (b, 0, 0, 0))],
        out_specs=(
            pl.BlockSpec((None, PLEN, NV, NP), lambda b: (b, 0, 0, 0)),
            pl.BlockSpec((None, NV, NP), lambda b: (b, 0, 0)),
            pl.BlockSpec((None, PLEN, NV, NP), lambda b: (b, 0, 0, 0)),
            pl.BlockSpec((None, NV, NP), lambda b: (b, 0, 0)),
        ),
        out_shape=(
            jax.ShapeDtypeStruct((bs, PLEN, NV, NP), jnp.float32),
            jax.ShapeDtypeStruct((bs, NV, NP), jnp.float32),
            jax.ShapeDtypeStruct((bs, PLEN, NV, NP), jnp.float32),
            jax.ShapeDtypeStruct((bs, NV, NP), jnp.float32),
        ),
        scratch_shapes=[pltpu.VMEM((NP, NP), jnp.bfloat16)],
        interpret=interpret,
    )(xt)


def kernel(x):
    xt = jnp.transpose(x, (0, 3, 2, 1))
    xtm, tm, xfm, fm = _run(xt)
    return (jnp.transpose(xtm, (0, 3, 2, 1)),
            jnp.transpose(tm, (0, 2, 1)),
            jnp.transpose(xfm, (0, 3, 2, 1)),
            jnp.transpose(fm, (0, 2, 1)))


# speedup vs baseline: 32.7328x; 1.9649x over previous
import jax
import jax.numpy as jnp
from jax.experimental import pallas as pl
from jax.experimental.pallas import tpu as pltpu

BS = 128
NP = 1024
NV = 16
PLEN = 12
K_TIME = 512
K_FREQ = 409
NB = 2


def _body(x_ref, xt_ref, tm_ref, xf_ref, fm_ref, ut_ref):
    b = pl.program_id(0)
    nr = 2 * NB * NV

    @pl.when(b == 0)
    def _init_ut():
        r = jax.lax.broadcasted_iota(jnp.int32, (NP, NP), 0)
        c = jax.lax.broadcasted_iota(jnp.int32, (NP, NP), 1)
        ut_ref[...] = (r < c).astype(jnp.bfloat16)

    xb = x_ref[...]
    s = jnp.sum(xb, axis=1)
    sq = jnp.sum(xb * xb, axis=1)
    mean = s * (1.0 / PLEN)
    var = (sq - s * mean) * (1.0 / (PLEN - 1))
    var = jnp.maximum(var, 0.0)
    cv = jnp.sqrt(var) / (mean + 1e-6)

    cv2 = cv.reshape(NB * NV, NP)
    bits1 = jax.lax.bitcast_convert_type(cv2, jnp.int32)
    bits = jnp.concatenate([bits1, bits1], axis=0)
    rowi = jax.lax.broadcasted_iota(jnp.int32, (nr, 1), 0)
    kvec = jnp.where(rowi < NB * NV, K_TIME, K_FREQ)
    kf = kvec.astype(jnp.float32)

    t = jnp.zeros((nr, 1), jnp.int32)
    for shift in range(30, -1, -1):
        cand = t | (1 << shift)
        ge = jnp.where(bits >= cand, 1.0, 0.0)
        cnt = jnp.sum(ge, axis=1, keepdims=True)
        t = jnp.where(cnt >= kf, cand, t)

    gt = bits > t
    eq = bits == t
    g = jnp.sum(jnp.where(gt, 1.0, 0.0), axis=1, keepdims=True)
    need = kf - g
    prefix = jnp.dot(eq.astype(jnp.bfloat16), ut_ref[...],
                     preferred_element_type=jnp.float32)
    tie = eq & (prefix < need)
    keep = 1.0 - (gt | tie).astype(jnp.float32)

    tmask = keep[:NB * NV].reshape(NB, NV, NP)
    fmask = keep[NB * NV:].reshape(NB, NV, NP)
    tm_ref[...] = tmask
    fm_ref[...] = fmask
    xt_ref[...] = xb * tmask[:, None]
    xf_ref[...] = xb * fmask[:, None]


def _run(xt, interpret=False):
    bs = xt.shape[0]
    return pl.pallas_call(
        _body,
        grid=(bs // NB,),
        in_specs=[pl.BlockSpec((NB, PLEN, NV, NP), lambda b: (b, 0, 0, 0))],
        out_specs=(
            pl.BlockSpec((NB, PLEN, NV, NP), lambda b: (b, 0, 0, 0)),
            pl.BlockSpec((NB, NV, NP), lambda b: (b, 0, 0)),
            pl.BlockSpec((NB, PLEN, NV, NP), lambda b: (b, 0, 0, 0)),
            pl.BlockSpec((NB, NV, NP), lambda b: (b, 0, 0)),
        ),
        out_shape=(
            jax.ShapeDtypeStruct((bs, PLEN, NV, NP), jnp.float32),
            jax.ShapeDtypeStruct((bs, NV, NP), jnp.float32),
            jax.ShapeDtypeStruct((bs, PLEN, NV, NP), jnp.float32),
            jax.ShapeDtypeStruct((bs, NV, NP), jnp.float32),
        ),
        scratch_shapes=[pltpu.VMEM((NP, NP), jnp.bfloat16)],
        interpret=interpret,
    )(xt)


def kernel(x):
    xt = jnp.transpose(x, (0, 3, 2, 1))
    xtm, tm, xfm, fm = _run(xt)
    return (jnp.transpose(xtm, (0, 3, 2, 1)),
            jnp.transpose(tm, (0, 2, 1)),
            jnp.transpose(xfm, (0, 3, 2, 1)),
            jnp.transpose(fm, (0, 2, 1)))


# speedup vs baseline: 55.6303x; 1.6995x over previous
import jax
import jax.numpy as jnp
from jax.experimental import pallas as pl
from jax.experimental.pallas import tpu as pltpu

BS = 128
NP = 1024
NV = 16
PLEN = 12
K_TIME = 512
K_FREQ = 409
NB = 4


def _body(x_ref, xt_ref, tm_ref, xf_ref, fm_ref, ut_ref):
    b = pl.program_id(0)
    nr = 2 * NB * NV

    @pl.when(b == 0)
    def _init_ut():
        r = jax.lax.broadcasted_iota(jnp.int32, (NP, NP), 0)
        c = jax.lax.broadcasted_iota(jnp.int32, (NP, NP), 1)
        ut_ref[...] = (r < c).astype(jnp.bfloat16)

    xb = x_ref[...]
    s = jnp.sum(xb, axis=1)
    sq = jnp.sum(xb * xb, axis=1)
    mean = s * (1.0 / PLEN)
    var = (sq - s * mean) * (1.0 / (PLEN - 1))
    var = jnp.maximum(var, 0.0)
    cv = jnp.sqrt(var) / (mean + 1e-6)

    cv2 = cv.reshape(NB * NV, NP)
    bits1 = jax.lax.bitcast_convert_type(cv2, jnp.int32)
    bits = jnp.concatenate([bits1, bits1], axis=0)
    rowi = jax.lax.broadcasted_iota(jnp.int32, (nr, 1), 0)
    kvec = jnp.where(rowi < NB * NV, K_TIME, K_FREQ)
    kf = kvec.astype(jnp.float32)

    t = jnp.zeros((nr, 1), jnp.int32)
    for shift in range(30, -1, -1):
        cand = t | (1 << shift)
        ge = jnp.where(bits >= cand, 1.0, 0.0)
        cnt = jnp.sum(ge, axis=1, keepdims=True)
        t = jnp.where(cnt >= kf, cand, t)

    gt = bits > t
    eq = bits == t
    g = jnp.sum(jnp.where(gt, 1.0, 0.0), axis=1, keepdims=True)
    need = kf - g
    prefix = jnp.dot(eq.astype(jnp.bfloat16), ut_ref[...],
                     preferred_element_type=jnp.float32)
    tie = eq & (prefix < need)
    keep = 1.0 - (gt | tie).astype(jnp.float32)

    tmask = keep[:NB * NV].reshape(NB, NV, NP)
    fmask = keep[NB * NV:].reshape(NB, NV, NP)
    tm_ref[...] = tmask
    fm_ref[...] = fmask
    xt_ref[...] = xb * tmask[:, None]
    xf_ref[...] = xb * fmask[:, None]


def _run(xt, interpret=False):
    bs = xt.shape[0]
    return pl.pallas_call(
        _body,
        grid=(bs // NB,),
        in_specs=[pl.BlockSpec((NB, PLEN, NV, NP), lambda b: (b, 0, 0, 0))],
        out_specs=(
            pl.BlockSpec((NB, PLEN, NV, NP), lambda b: (b, 0, 0, 0)),
            pl.BlockSpec((NB, NV, NP), lambda b: (b, 0, 0)),
            pl.BlockSpec((NB, PLEN, NV, NP), lambda b: (b, 0, 0, 0)),
            pl.BlockSpec((NB, NV, NP), lambda b: (b, 0, 0)),
        ),
        out_shape=(
            jax.ShapeDtypeStruct((bs, PLEN, NV, NP), jnp.float32),
            jax.ShapeDtypeStruct((bs, NV, NP), jnp.float32),
            jax.ShapeDtypeStruct((bs, PLEN, NV, NP), jnp.float32),
            jax.ShapeDtypeStruct((bs, NV, NP), jnp.float32),
        ),
        scratch_shapes=[pltpu.VMEM((NP, NP), jnp.bfloat16)],
        interpret=interpret,
    )(xt)


def kernel(x):
    xt = jnp.transpose(x, (0, 3, 2, 1))
    xtm, tm, xfm, fm = _run(xt)
    return (jnp.transpose(xtm, (0, 3, 2, 1)),
            jnp.transpose(tm, (0, 2, 1)),
            jnp.transpose(xfm, (0, 3, 2, 1)),
            jnp.transpose(fm, (0, 2, 1)))


# speedup vs baseline: 69.0199x; 1.2407x over previous
import jax
import jax.numpy as jnp
from jax.experimental import pallas as pl
from jax.experimental.pallas import tpu as pltpu

BS = 128
NP = 1024
NV = 16
PLEN = 12
K_TIME = 512
K_FREQ = 409
NB = 8


def _body(x_ref, xt_ref, tm_ref, xf_ref, fm_ref, ut_ref):
    b = pl.program_id(0)
    nr = 2 * NB * NV

    @pl.when(b == 0)
    def _init_ut():
        r = jax.lax.broadcasted_iota(jnp.int32, (NP, NP), 0)
        c = jax.lax.broadcasted_iota(jnp.int32, (NP, NP), 1)
        ut_ref[...] = (r < c).astype(jnp.bfloat16)

    xb = x_ref[...]
    s = jnp.sum(xb, axis=1)
    sq = jnp.sum(xb * xb, axis=1)
    mean = s * (1.0 / PLEN)
    var = (sq - s * mean) * (1.0 / (PLEN - 1))
    var = jnp.maximum(var, 0.0)
    cv = jnp.sqrt(var) / (mean + 1e-6)

    cv2 = cv.reshape(NB * NV, NP)
    bits1 = jax.lax.bitcast_convert_type(cv2, jnp.int32)
    bits = jnp.concatenate([bits1, bits1], axis=0)
    rowi = jax.lax.broadcasted_iota(jnp.int32, (nr, 1), 0)
    kvec = jnp.where(rowi < NB * NV, K_TIME, K_FREQ)
    kf = kvec.astype(jnp.float32)

    t = jnp.zeros((nr, 1), jnp.int32)
    for shift in range(30, -1, -1):
        cand = t | (1 << shift)
        ge = jnp.where(bits >= cand, 1.0, 0.0)
        cnt = jnp.sum(ge, axis=1, keepdims=True)
        t = jnp.where(cnt >= kf, cand, t)

    gt = bits > t
    eq = bits == t
    g = jnp.sum(jnp.where(gt, 1.0, 0.0), axis=1, keepdims=True)
    need = kf - g
    prefix = jnp.dot(eq.astype(jnp.bfloat16), ut_ref[...],
                     preferred_element_type=jnp.float32)
    tie = eq & (prefix < need)
    keep = 1.0 - (gt | tie).astype(jnp.float32)

    tmask = keep[:NB * NV].reshape(NB, NV, NP)
    fmask = keep[NB * NV:].reshape(NB, NV, NP)
    tm_ref[...] = tmask
    fm_ref[...] = fmask
    xt_ref[...] = xb * tmask[:, None]
    xf_ref[...] = xb * fmask[:, None]


def _run(xt, interpret=False):
    bs = xt.shape[0]
    return pl.pallas_call(
        _body,
        grid=(bs // NB,),
        in_specs=[pl.BlockSpec((NB, PLEN, NV, NP), lambda b: (b, 0, 0, 0))],
        out_specs=(
            pl.BlockSpec((NB, PLEN, NV, NP), lambda b: (b, 0, 0, 0)),
            pl.BlockSpec((NB, NV, NP), lambda b: (b, 0, 0)),
            pl.BlockSpec((NB, PLEN, NV, NP), lambda b: (b, 0, 0, 0)),
            pl.BlockSpec((NB, NV, NP), lambda b: (b, 0, 0)),
        ),
        out_shape=(
            jax.ShapeDtypeStruct((bs, PLEN, NV, NP), jnp.float32),
            jax.ShapeDtypeStruct((bs, NV, NP), jnp.float32),
            jax.ShapeDtypeStruct((bs, PLEN, NV, NP), jnp.float32),
            jax.ShapeDtypeStruct((bs, NV, NP), jnp.float32),
        ),
        scratch_shapes=[pltpu.VMEM((NP, NP), jnp.bfloat16)],
        interpret=interpret,
    )(xt)


def kernel(x):
    xt = jnp.transpose(x, (0, 3, 2, 1))
    xtm, tm, xfm, fm = _run(xt)
    return (jnp.transpose(xtm, (0, 3, 2, 1)),
            jnp.transpose(tm, (0, 2, 1)),
            jnp.transpose(xfm, (0, 3, 2, 1)),
            jnp.transpose(fm, (0, 2, 1)))
